# baseline (speedup 1.0000x reference)
import functools

import jax
import jax.numpy as jnp
from jax import lax
from jax.experimental import pallas as pl
from jax.experimental.pallas import tpu as pltpu
from jax.experimental.pallas import tpu_sc as plsc

N_NODES = 10000
E = 320000
R = 8
HID = 128

NC = 2
NS = 16
NW = NC * NS
EPW = E // NW
SUB = 80
NSUB = 5
CHUNK = SUB * NSUB
NCHUNK = EPW // CHUNK
CNT_PAD = 632 * 128
CNT_PER_TILE = CNT_PAD // NS
ACC_ROWS = 10240
EVAC = N_NODES // NS

_mesh = plsc.VectorSubcoreMesh(core_axis_name="c", subcore_axis_name="s")


def _counts_body(cidx_hbm, ones_hbm, zeros_hbm, out_hbm, ci1_v, ci2_v,
                 ones_v, zb_v, acc):
    c = lax.axis_index("c")
    s = lax.axis_index("s")
    wid = s * NC + c
    pltpu.sync_copy(zeros_hbm, zb_v)
    pltpu.sync_copy(zb_v,
                    acc.at[pl.ds(pl.multiple_of(s * CNT_PER_TILE, 8),
                                 CNT_PER_TILE)])
    pltpu.sync_copy(ones_hbm, ones_v)
    plsc.subcore_barrier()
    for k in range(NCHUNK):
        base = pl.multiple_of(wid * EPW + k * CHUNK, 8)
        pltpu.sync_copy(cidx_hbm.at[pl.ds(base, CHUNK)], ci1_v)
        for j in range(NSUB):
            for i in range(SUB // 16):
                ci2_v[j, pl.ds(i * 16, 16)] = ci1_v[pl.ds(j * SUB + i * 16,
                                                          16)]
        for j in range(NSUB):
            pltpu.sync_copy(ones_v, acc.at[ci2_v.at[j]], add=True)
    plsc.subcore_barrier()
    pltpu.sync_copy(
        acc.at[pl.ds(pl.multiple_of(s * CNT_PER_TILE, 8), CNT_PER_TILE)],
        zb_v)
    pltpu.sync_copy(
        zb_v,
        out_hbm.at[pl.ds(pl.multiple_of(c * CNT_PAD + s * CNT_PER_TILE, 8),
                         CNT_PER_TILE)])


_counts_call = functools.partial(
    pl.kernel,
    _counts_body,
    out_type=jax.ShapeDtypeStruct((NC * CNT_PAD,), jnp.float32),
    mesh=_mesh,
    compiler_params=pltpu.CompilerParams(needs_layout_passes=False),
    scratch_types=[
        pltpu.VMEM((CHUNK,), jnp.int32),
        pltpu.VMEM((NSUB, SUB), jnp.int32),
        pltpu.VMEM((SUB,), jnp.float32),
        pltpu.VMEM((CNT_PER_TILE,), jnp.float32),
        pltpu.VMEM_SHARED((CNT_PAD,), jnp.float32),
    ],
)()


NAGG = EPW // SUB


NBUF = 4


def _agg_body(hz9_hbm, gcidx_hbm, dst_hbm, inv_hbm, zeros_hbm,
              p0_hbm, p1_hbm, *bufs):
    gc = bufs[0:4]
    dv = bufs[8:12]
    iv = bufs[12:16]
    rw = bufs[16:20]
    acc = bufs[20]
    xsem = bufs[21:25]
    gsem = bufs[25:29]
    isem = bufs[29:33]
    dsem = bufs[33:37]
    ssem = bufs[37:41]
    gi = tuple(g.at[pl.ds(0, SUB)] for g in gc)
    ci = tuple(g.at[pl.ds(SUB, SUB)] for g in gc)
    rw0 = rw[0]
    c = lax.axis_index("c")
    s = lax.axis_index("s")
    wid = s * NC + c
    rpt = ACC_ROWS // NS
    pltpu.sync_copy(zeros_hbm, rw0)
    for h in range(rpt // SUB):
        pltpu.sync_copy(
            rw0, acc.at[pl.ds(pl.multiple_of(s * rpt + h * SUB, 8), SUB)])
    plsc.subcore_barrier()

    def base_of(k):
        return pl.multiple_of(wid * EPW + k * SUB, 8)

    def gica(k, b, sem):
        bs = pl.multiple_of(wid * EPW * 2 + k * (2 * SUB), 8)
        return (pltpu.make_async_copy(gcidx_hbm.at[pl.ds(bs, 2 * SUB)],
                                      gc[b], sem),)

    def dca(k, b, sem):
        return pltpu.make_async_copy(dst_hbm.at[pl.ds(base_of(k), SUB)],
                                     dv[b], sem)

    def gca(b, sem):
        return pltpu.make_async_copy(hz9_hbm.at[gi[b]], rw[b], sem)

    def ica(b, sem):
        return pltpu.make_async_copy(inv_hbm.at[ci[b]], iv[b], sem)

    def sca_start(b, sem):
        pltpu.async_copy(rw[b], acc.at[dv[b]], sem, add=True)

    def sca(b, sem):
        return pltpu.make_async_copy(rw[b], acc.at[dv[b]], sem)

    pltpu.sync_copy(gcidx_hbm.at[pl.ds(pl.multiple_of(wid * EPW * 2, 8),
                                       2 * SUB)], gc[0])
    pltpu.sync_copy(dst_hbm.at[pl.ds(base_of(0), SUB)], dv[0])
    gca(0, gsem[0]).start()
    ica(0, isem[0]).start()
    for cp in gica(1, 1, xsem[1]):
        cp.start()
    for cp in gica(1, 1, xsem[1]):
        cp.wait()
    gca(1, gsem[1]).start()
    ica(1, isem[1]).start()
    dca(1, 1, dsem[1]).start()
    for cp in gica(2, 2, xsem[2]):
        cp.start()

    def chunk_body(k, carry):
        sb = lax.rem(k, NBUF)
        for b in range(NBUF):
            b1 = (b + 1) % NBUF
            b2 = (b + 2) % NBUF
            b3 = (b + 3) % NBUF

            @pl.when(sb == b)
            def _(b=b, b1=b1, b2=b2, b3=b3):
                @pl.when(k + 2 < NAGG)
                def _():
                    for cp in gica(k + 2, b2, xsem[b2]):
                        cp.wait()

                    @pl.when(k >= 2)
                    def _():
                        sca(b2, ssem[b2]).wait()

                    gca(b2, gsem[b2]).start()
                    ica(b2, isem[b2]).start()
                    dca(k + 2, b2, dsem[b2]).start()

                @pl.when(k + 3 < NAGG)
                def _():
                    for cp in gica(k + 3, b3, xsem[b3]):
                        cp.start()

                gca(b, gsem[b]).wait()
                ica(b, isem[b]).wait()

                @pl.when(k >= 1)
                def _():
                    dca(k, b, dsem[b]).wait()

                @plsc.parallel_loop(0, SUB, unroll=8)
                def _(i, b=b):
                    bc = plsc.load_gather(
                        iv[b], [jnp.full((16,), 0, jnp.int32) + i])
                    for v in range(HID // 16):
                        sl = pl.ds(v * 16, 16)
                        rw[b][i, sl] = rw[b][i, sl] * bc

                sca_start(b, ssem[b])
        return carry

    lax.fori_loop(0, NAGG, chunk_body, 0)
    for j in range(NAGG - 4, NAGG):
        if j >= 0:
            sca(j % NBUF, ssem[j % NBUF]).wait()
    plsc.subcore_barrier()
    for h in range(rpt // SUB):
        evac = pl.ds(pl.multiple_of(s * rpt + h * SUB, 8), SUB)
        hb = rw[h % 2]
        pltpu.sync_copy(acc.at[evac], hb)

        @pl.when(c == 0)
        def _(hb=hb):
            pltpu.sync_copy(hb, p0_hbm.at[evac])

        @pl.when(c == 1)
        def _(hb=hb):
            pltpu.sync_copy(hb, p1_hbm.at[evac])


_agg_call = functools.partial(
    pl.kernel,
    _agg_body,
    out_type=(jax.ShapeDtypeStruct((ACC_ROWS, HID), jnp.float32),
              jax.ShapeDtypeStruct((ACC_ROWS, HID), jnp.float32)),
    mesh=_mesh,
    compiler_params=pltpu.CompilerParams(needs_layout_passes=False),
    scratch_types=(
        [pltpu.VMEM((2 * SUB,), jnp.int32)] * 4
        + [pltpu.VMEM((SUB,), jnp.int32)] * 8
        + [pltpu.VMEM((SUB,), jnp.float32)] * 4
        + [pltpu.VMEM((SUB, HID), jnp.float32)] * 4
        + [pltpu.VMEM_SHARED((ACC_ROWS, HID), jnp.float32)]
        + [pltpu.SemaphoreType.DMA] * 20
    ),
)()


def _proj_body(xa_ref, xb_ref, wa_ref, ba_ref, wb_ref, bb_ref, out_ref):
    i = pl.program_id(0)

    @pl.when(i < 5)
    def _():
        out_ref[...] = jnp.maximum(
            jnp.dot(xa_ref[...], wa_ref[...],
                    preferred_element_type=jnp.float32) + ba_ref[...], 0.0)

    @pl.when(i >= 5)
    def _():
        out_ref[...] = jnp.maximum(
            jnp.dot(xb_ref[...], wb_ref[...],
                    preferred_element_type=jnp.float32) + bb_ref[...], 0.0)


def _proj(x_a, Wa, ba, x_b, Wb, bb):
    blk = 1000
    return pl.pallas_call(
        _proj_body,
        grid=(10,),
        in_specs=[
            pl.BlockSpec((blk, 128), lambda i: (jnp.minimum(i, 4), 0)),
            pl.BlockSpec((blk, 256), lambda i: (jnp.maximum(i - 5, 0), 0)),
            pl.BlockSpec((128, HID), lambda i: (0, 0)),
            pl.BlockSpec((1, HID), lambda i: (0, 0)),
            pl.BlockSpec((256, HID), lambda i: (0, 0)),
            pl.BlockSpec((1, HID), lambda i: (0, 0)),
        ],
        out_specs=pl.BlockSpec((blk, HID), lambda i: (i, 0)),
        out_shape=jax.ShapeDtypeStruct((N_NODES, HID), jnp.float32),
    )(x_a, x_b, Wa, ba.reshape(1, HID), Wb, bb.reshape(1, HID))


def _recip_body(cp_ref, out_ref):
    tot = cp_ref[0] + cp_ref[1]
    out_ref[...] = 1.0 / jnp.maximum(tot, 1.0)


def _recip(counts_p):
    return pl.pallas_call(
        _recip_body,
        out_shape=jax.ShapeDtypeStruct((CNT_PAD // 128, 128), jnp.float32),
    )(counts_p.reshape(NC, CNT_PAD // 128, 128)).reshape(CNT_PAD)


def _wprep_body(basis_ref, comp_ref, root_ref, out_ref):
    li = pl.program_id(0)
    for r in range(R):
        acc = basis_ref[0, 0] * comp_ref[li, r, 0]
        for b in range(1, 4):
            acc = acc + basis_ref[0, b] * comp_ref[li, r, b]
        out_ref[0, :, r * HID:(r + 1) * HID] = acc
    out_ref[0, :, R * HID:] = root_ref[0]


def _wprep3(basis, comp, root):
    return pl.pallas_call(
        _wprep_body,
        grid=(3,),
        in_specs=[
            pl.BlockSpec((1, 4, HID, HID), lambda i: (i, 0, 0, 0)),
            pl.BlockSpec(memory_space=pltpu.SMEM),
            pl.BlockSpec((1, HID, HID), lambda i: (i, 0, 0)),
        ],
        out_specs=pl.BlockSpec((1, HID, (R + 1) * HID), lambda i: (i, 0, 0)),
        out_shape=jax.ShapeDtypeStruct((3, HID, (R + 1) * HID), jnp.float32),
    )(basis, comp, root)


def _mm0_body(x_ref, wf_ref, bf_ref, out_ref):
    out_ref[...] = jnp.dot(x_ref[...], wf_ref[...],
                           preferred_element_type=jnp.float32) + bf_ref[...]


def _mmn_body(z_ref, p0_ref, p1_ref, wf_ref, bf_ref, out_ref):
    x = jnp.maximum(z_ref[...] + p0_ref[...] + p1_ref[...], 0.0)
    out_ref[...] = jnp.dot(x, wf_ref[...],
                           preferred_element_type=jnp.float32) + bf_ref[...]


_MMBLK = 1000
_WCOLS = (R + 1) * HID


def _mm0(x, wf, bf):
    return pl.pallas_call(
        _mm0_body,
        grid=(N_NODES // _MMBLK,),
        in_specs=[
            pl.BlockSpec((_MMBLK, HID), lambda i: (i, 0)),
            pl.BlockSpec((HID, _WCOLS), lambda i: (0, 0)),
            pl.BlockSpec((1, _WCOLS), lambda i: (0, 0)),
        ],
        out_specs=pl.BlockSpec((_MMBLK, _WCOLS), lambda i: (i, 0)),
        out_shape=jax.ShapeDtypeStruct((N_NODES, _WCOLS), jnp.float32),
    )(x, wf, bf)


def _mmn(hz_prev, p0, p1, wf, bf):
    return pl.pallas_call(
        _mmn_body,
        grid=(N_NODES // _MMBLK,),
        in_specs=[
            pl.BlockSpec((_MMBLK, HID), lambda i: (i, R)),
            pl.BlockSpec((_MMBLK, HID), lambda i: (i, 0)),
            pl.BlockSpec((_MMBLK, HID), lambda i: (i, 0)),
            pl.BlockSpec((HID, _WCOLS), lambda i: (0, 0)),
            pl.BlockSpec((1, _WCOLS), lambda i: (0, 0)),
        ],
        out_specs=pl.BlockSpec((_MMBLK, _WCOLS), lambda i: (i, 0)),
        out_shape=jax.ShapeDtypeStruct((N_NODES, _WCOLS), jnp.float32),
    )(hz_prev, p0, p1, wf, bf)


def _fin_body(z_ref, p0_ref, p1_ref, out_ref):
    out_ref[...] = z_ref[...] + p0_ref[...] + p1_ref[...]


def _fin(hz_prev, p0, p1):
    return pl.pallas_call(
        _fin_body,
        grid=(N_NODES // _MMBLK,),
        in_specs=[
            pl.BlockSpec((_MMBLK, HID), lambda i: (i, R)),
            pl.BlockSpec((_MMBLK, HID), lambda i: (i, 0)),
            pl.BlockSpec((_MMBLK, HID), lambda i: (i, 0)),
        ],
        out_specs=pl.BlockSpec((_MMBLK, HID), lambda i: (i, 0)),
        out_shape=jax.ShapeDtypeStruct((N_NODES, HID), jnp.float32),
    )(hz_prev, p0, p1)


def kernel(x_a, x_b, edge_index, edge_type, Wa, ba, Wb, bb,
           basis0, comp0, root0, bias0,
           basis1, comp1, root1, bias1,
           basis2, comp2, root2, bias2):
    src = edge_index[0]
    dst = edge_index[1]
    et = edge_type
    gidx = src * 9 + et
    cidx = dst * 8 + et
    gcidx = jnp.concatenate([gidx.reshape(-1, 1, SUB),
                             cidx.reshape(-1, 1, SUB)],
                            axis=1).reshape(-1)
    dst2 = dst

    ones_hbm = jnp.ones((SUB,), jnp.float32)
    zeros1_hbm = jnp.zeros((CNT_PER_TILE,), jnp.float32)
    zeros2_hbm = jnp.zeros((SUB, HID), jnp.float32)

    counts_p = _counts_call(cidx, ones_hbm, zeros1_hbm)
    inv_dr = _recip(counts_p)

    x0 = _proj(x_a, Wa, ba, x_b, Wb, bb)

    def bias_full(bias):
        return jnp.concatenate(
            [jnp.zeros((R * HID,), jnp.float32), bias]).reshape(1, _WCOLS)

    wf = _wprep3(jnp.stack([basis0, basis1, basis2]),
                 jnp.stack([comp0, comp1, comp2]),
                 jnp.stack([root0, root1, root2]))
    wf0, wf1, wf2 = wf[0], wf[1], wf[2]

    hz0 = _mm0(x0, wf0, bias_full(bias0))
    p0a, p0b = _agg_call(hz0.reshape(N_NODES * 9, HID), gcidx, dst2,
                         inv_dr, zeros2_hbm)
    hz1 = _mmn(hz0, p0a, p0b, wf1, bias_full(bias1))
    p1a, p1b = _agg_call(hz1.reshape(N_NODES * 9, HID), gcidx, dst2,
                         inv_dr, zeros2_hbm)
    hz2 = _mmn(hz1, p1a, p1b, wf2, bias_full(bias2))
    p2a, p2b = _agg_call(hz2.reshape(N_NODES * 9, HID), gcidx, dst2,
                         inv_dr, zeros2_hbm)
    return _fin(hz2, p2a, p2b)


# speedup vs baseline: 1.0441x; 1.0441x over previous
import functools

import jax
import jax.numpy as jnp
from jax import lax
from jax.experimental import pallas as pl
from jax.experimental.pallas import tpu as pltpu
from jax.experimental.pallas import tpu_sc as plsc

N_NODES = 10000
E = 320000
R = 8
HID = 128

NC = 2
NS = 16
NW = NC * NS
EPW = E // NW
SUB = 80
NSUB = 5
CHUNK = SUB * NSUB
NCHUNK = EPW // CHUNK
CNT_PAD = 632 * 128
CNT_PER_TILE = CNT_PAD // NS
ACC_ROWS = 10240
EVAC = N_NODES // NS

_mesh = plsc.VectorSubcoreMesh(core_axis_name="c", subcore_axis_name="s")


def _counts_body(cidx_hbm, ones_hbm, zeros_hbm, out_hbm, ci1_v, ci2_v,
                 ones_v, zb_v, acc):
    c = lax.axis_index("c")
    s = lax.axis_index("s")
    wid = s * NC + c
    pltpu.sync_copy(zeros_hbm, zb_v)
    pltpu.sync_copy(zb_v,
                    acc.at[pl.ds(pl.multiple_of(s * CNT_PER_TILE, 8),
                                 CNT_PER_TILE)])
    pltpu.sync_copy(ones_hbm, ones_v)
    plsc.subcore_barrier()
    for k in range(NCHUNK):
        base = pl.multiple_of(wid * EPW + k * CHUNK, 8)
        pltpu.sync_copy(cidx_hbm.at[pl.ds(base, CHUNK)], ci1_v)
        for j in range(NSUB):
            for i in range(SUB // 16):
                ci2_v[j, pl.ds(i * 16, 16)] = ci1_v[pl.ds(j * SUB + i * 16,
                                                          16)]
        for j in range(NSUB):
            pltpu.sync_copy(ones_v, acc.at[ci2_v.at[j]], add=True)
    plsc.subcore_barrier()
    pltpu.sync_copy(
        acc.at[pl.ds(pl.multiple_of(s * CNT_PER_TILE, 8), CNT_PER_TILE)],
        zb_v)
    pltpu.sync_copy(
        zb_v,
        out_hbm.at[pl.ds(pl.multiple_of(c * CNT_PAD + s * CNT_PER_TILE, 8),
                         CNT_PER_TILE)])


_counts_call = functools.partial(
    pl.kernel,
    _counts_body,
    out_type=jax.ShapeDtypeStruct((NC * CNT_PAD,), jnp.float32),
    mesh=_mesh,
    compiler_params=pltpu.CompilerParams(needs_layout_passes=False),
    scratch_types=[
        pltpu.VMEM((CHUNK,), jnp.int32),
        pltpu.VMEM((NSUB, SUB), jnp.int32),
        pltpu.VMEM((SUB,), jnp.float32),
        pltpu.VMEM((CNT_PER_TILE,), jnp.float32),
        pltpu.VMEM_SHARED((CNT_PAD,), jnp.float32),
    ],
)()


NAGG = EPW // SUB


NBUF = 4


def _agg_body(hz9_hbm, gidx_hbm, cidx_hbm, dst_hbm, inv_hbm, zeros_hbm,
              p0_hbm, p1_hbm, *bufs):
    gi = bufs[0:4]
    ci = bufs[4:8]
    dv = bufs[8:12]
    iv = bufs[12:16]
    rw = bufs[16:20]
    acc = bufs[20]
    xsem = bufs[21:25]
    gsem = bufs[25:29]
    isem = bufs[29:33]
    dsem = bufs[33:37]
    ssem = bufs[37:41]
    rw0 = rw[0]
    c = lax.axis_index("c")
    s = lax.axis_index("s")
    wid = s * NC + c
    rpt = ACC_ROWS // NS
    pltpu.sync_copy(zeros_hbm, rw0)
    for h in range(rpt // SUB):
        pltpu.sync_copy(
            rw0, acc.at[pl.ds(pl.multiple_of(s * rpt + h * SUB, 8), SUB)])
    plsc.subcore_barrier()

    def base_of(k):
        return pl.multiple_of(wid * EPW + k * SUB, 8)

    def gica(k, b, sem):
        bs = base_of(k)
        return (pltpu.make_async_copy(gidx_hbm.at[pl.ds(bs, SUB)], gi[b], sem),
                pltpu.make_async_copy(cidx_hbm.at[pl.ds(bs, SUB)], ci[b], sem))

    def dca(k, b, sem):
        return pltpu.make_async_copy(dst_hbm.at[pl.ds(base_of(k), SUB)],
                                     dv[b], sem)

    def gca(b, sem):
        return pltpu.make_async_copy(hz9_hbm.at[gi[b]], rw[b], sem)

    def ica(b, sem):
        return pltpu.make_async_copy(inv_hbm.at[ci[b]], iv[b], sem)

    def sca_start(b, sem):
        pltpu.async_copy(rw[b], acc.at[dv[b]], sem, add=True)

    def sca(b, sem):
        return pltpu.make_async_copy(rw[b], acc.at[dv[b]], sem)

    pltpu.sync_copy(gidx_hbm.at[pl.ds(base_of(0), SUB)], gi[0])
    pltpu.sync_copy(cidx_hbm.at[pl.ds(base_of(0), SUB)], ci[0])
    pltpu.sync_copy(dst_hbm.at[pl.ds(base_of(0), SUB)], dv[0])
    gca(0, gsem[0]).start()
    ica(0, isem[0]).start()
    for cp in gica(1, 1, xsem[1]):
        cp.start()
    for cp in gica(1, 1, xsem[1]):
        cp.wait()
    gca(1, gsem[1]).start()
    ica(1, isem[1]).start()
    dca(1, 1, dsem[1]).start()
    for cp in gica(2, 2, xsem[2]):
        cp.start()

    def chunk_body(k, carry):
        sb = lax.rem(k, NBUF)
        for b in range(NBUF):
            b1 = (b + 1) % NBUF
            b2 = (b + 2) % NBUF
            b3 = (b + 3) % NBUF

            @pl.when(sb == b)
            def _(b=b, b1=b1, b2=b2, b3=b3):
                @pl.when(k + 2 < NAGG)
                def _():
                    for cp in gica(k + 2, b2, xsem[b2]):
                        cp.wait()

                    @pl.when(k >= 2)
                    def _():
                        sca(b2, ssem[b2]).wait()

                    gca(b2, gsem[b2]).start()
                    ica(b2, isem[b2]).start()
                    dca(k + 2, b2, dsem[b2]).start()

                @pl.when(k + 3 < NAGG)
                def _():
                    for cp in gica(k + 3, b3, xsem[b3]):
                        cp.start()

                gca(b, gsem[b]).wait()
                ica(b, isem[b]).wait()

                @pl.when(k >= 1)
                def _():
                    dca(k, b, dsem[b]).wait()

                @plsc.parallel_loop(0, SUB, unroll=8)
                def _(i, b=b):
                    bc = plsc.load_gather(
                        iv[b], [jnp.full((16,), 0, jnp.int32) + i])
                    for v in range(HID // 16):
                        sl = pl.ds(v * 16, 16)
                        rw[b][i, sl] = rw[b][i, sl] * bc

                sca_start(b, ssem[b])
        return carry

    lax.fori_loop(0, NAGG, chunk_body, 0)
    for j in range(NAGG - 4, NAGG):
        if j >= 0:
            sca(j % NBUF, ssem[j % NBUF]).wait()
    plsc.subcore_barrier()
    for h in range(rpt // SUB):
        evac = pl.ds(pl.multiple_of(s * rpt + h * SUB, 8), SUB)
        hb = rw[h % 2]
        pltpu.sync_copy(acc.at[evac], hb)

        @pl.when(c == 0)
        def _(hb=hb):
            pltpu.sync_copy(hb, p0_hbm.at[evac])

        @pl.when(c == 1)
        def _(hb=hb):
            pltpu.sync_copy(hb, p1_hbm.at[evac])


_agg_call = functools.partial(
    pl.kernel,
    _agg_body,
    out_type=(jax.ShapeDtypeStruct((ACC_ROWS, HID), jnp.float32),
              jax.ShapeDtypeStruct((ACC_ROWS, HID), jnp.float32)),
    mesh=_mesh,
    compiler_params=pltpu.CompilerParams(needs_layout_passes=False),
    scratch_types=(
        [pltpu.VMEM((SUB,), jnp.int32)] * 12
        + [pltpu.VMEM((SUB,), jnp.float32)] * 4
        + [pltpu.VMEM((SUB, HID), jnp.float32)] * 4
        + [pltpu.VMEM_SHARED((ACC_ROWS, HID), jnp.float32)]
        + [pltpu.SemaphoreType.DMA] * 20
    ),
)()


def _proj_body(xa_ref, xb_ref, wa_ref, ba_ref, wb_ref, bb_ref, out_ref):
    i = pl.program_id(0)

    @pl.when(i < 5)
    def _():
        out_ref[...] = jnp.maximum(
            jnp.dot(xa_ref[...], wa_ref[...],
                    preferred_element_type=jnp.float32) + ba_ref[...], 0.0)

    @pl.when(i >= 5)
    def _():
        out_ref[...] = jnp.maximum(
            jnp.dot(xb_ref[...], wb_ref[...],
                    preferred_element_type=jnp.float32) + bb_ref[...], 0.0)


def _proj(x_a, Wa, ba, x_b, Wb, bb):
    blk = 1000
    return pl.pallas_call(
        _proj_body,
        grid=(10,),
        in_specs=[
            pl.BlockSpec((blk, 128), lambda i: (jnp.minimum(i, 4), 0)),
            pl.BlockSpec((blk, 256), lambda i: (jnp.maximum(i - 5, 0), 0)),
            pl.BlockSpec((128, HID), lambda i: (0, 0)),
            pl.BlockSpec((1, HID), lambda i: (0, 0)),
            pl.BlockSpec((256, HID), lambda i: (0, 0)),
            pl.BlockSpec((1, HID), lambda i: (0, 0)),
        ],
        out_specs=pl.BlockSpec((blk, HID), lambda i: (i, 0)),
        out_shape=jax.ShapeDtypeStruct((N_NODES, HID), jnp.float32),
    )(x_a, x_b, Wa, ba.reshape(1, HID), Wb, bb.reshape(1, HID))


def _recip_body(cp_ref, out_ref):
    tot = cp_ref[0] + cp_ref[1]
    out_ref[...] = 1.0 / jnp.maximum(tot, 1.0)


def _recip(counts_p):
    return pl.pallas_call(
        _recip_body,
        out_shape=jax.ShapeDtypeStruct((CNT_PAD // 128, 128), jnp.float32),
    )(counts_p.reshape(NC, CNT_PAD // 128, 128)).reshape(CNT_PAD)


def _wprep_body(basis_ref, comp_ref, root_ref, out_ref):
    li = pl.program_id(0)
    for r in range(R):
        acc = basis_ref[0, 0] * comp_ref[li, r, 0]
        for b in range(1, 4):
            acc = acc + basis_ref[0, b] * comp_ref[li, r, b]
        out_ref[0, :, r * HID:(r + 1) * HID] = acc
    out_ref[0, :, R * HID:] = root_ref[0]


def _wprep3(basis, comp, root):
    return pl.pallas_call(
        _wprep_body,
        grid=(3,),
        in_specs=[
            pl.BlockSpec((1, 4, HID, HID), lambda i: (i, 0, 0, 0)),
            pl.BlockSpec(memory_space=pltpu.SMEM),
            pl.BlockSpec((1, HID, HID), lambda i: (i, 0, 0)),
        ],
        out_specs=pl.BlockSpec((1, HID, (R + 1) * HID), lambda i: (i, 0, 0)),
        out_shape=jax.ShapeDtypeStruct((3, HID, (R + 1) * HID), jnp.float32),
    )(basis, comp, root)


def _mm0_body(x_ref, wf_ref, bf_ref, out_ref):
    out_ref[...] = jnp.dot(x_ref[...], wf_ref[...],
                           preferred_element_type=jnp.float32) + bf_ref[...]


def _mmn_body(z_ref, p0_ref, p1_ref, wf_ref, bf_ref, out_ref):
    x = jnp.maximum(z_ref[...] + p0_ref[...] + p1_ref[...], 0.0)
    out_ref[...] = jnp.dot(x, wf_ref[...],
                           preferred_element_type=jnp.float32) + bf_ref[...]


_MMBLK = 1000
_WCOLS = (R + 1) * HID


def _mm0(x, wf, bf):
    return pl.pallas_call(
        _mm0_body,
        grid=(N_NODES // _MMBLK,),
        in_specs=[
            pl.BlockSpec((_MMBLK, HID), lambda i: (i, 0)),
            pl.BlockSpec((HID, _WCOLS), lambda i: (0, 0)),
            pl.BlockSpec((1, _WCOLS), lambda i: (0, 0)),
        ],
        out_specs=pl.BlockSpec((_MMBLK, _WCOLS), lambda i: (i, 0)),
        out_shape=jax.ShapeDtypeStruct((N_NODES, _WCOLS), jnp.float32),
    )(x, wf, bf)


def _mmn(hz_prev, p0, p1, wf, bf):
    return pl.pallas_call(
        _mmn_body,
        grid=(N_NODES // _MMBLK,),
        in_specs=[
            pl.BlockSpec((_MMBLK, HID), lambda i: (i, R)),
            pl.BlockSpec((_MMBLK, HID), lambda i: (i, 0)),
            pl.BlockSpec((_MMBLK, HID), lambda i: (i, 0)),
            pl.BlockSpec((HID, _WCOLS), lambda i: (0, 0)),
            pl.BlockSpec((1, _WCOLS), lambda i: (0, 0)),
        ],
        out_specs=pl.BlockSpec((_MMBLK, _WCOLS), lambda i: (i, 0)),
        out_shape=jax.ShapeDtypeStruct((N_NODES, _WCOLS), jnp.float32),
    )(hz_prev, p0, p1, wf, bf)


def _fin_body(z_ref, p0_ref, p1_ref, out_ref):
    out_ref[...] = z_ref[...] + p0_ref[...] + p1_ref[...]


def _fin(hz_prev, p0, p1):
    return pl.pallas_call(
        _fin_body,
        grid=(N_NODES // _MMBLK,),
        in_specs=[
            pl.BlockSpec((_MMBLK, HID), lambda i: (i, R)),
            pl.BlockSpec((_MMBLK, HID), lambda i: (i, 0)),
            pl.BlockSpec((_MMBLK, HID), lambda i: (i, 0)),
        ],
        out_specs=pl.BlockSpec((_MMBLK, HID), lambda i: (i, 0)),
        out_shape=jax.ShapeDtypeStruct((N_NODES, HID), jnp.float32),
    )(hz_prev, p0, p1)


def kernel(x_a, x_b, edge_index, edge_type, Wa, ba, Wb, bb,
           basis0, comp0, root0, bias0,
           basis1, comp1, root1, bias1,
           basis2, comp2, root2, bias2):
    src = edge_index[0]
    dst = edge_index[1]
    et = edge_type
    gidx = src * 9 + et
    cidx = dst * 8 + et
    dst2 = dst

    ones_hbm = jnp.ones((SUB,), jnp.float32)
    zeros1_hbm = jnp.zeros((CNT_PER_TILE,), jnp.float32)
    zeros2_hbm = jnp.zeros((SUB, HID), jnp.float32)

    counts_p = _counts_call(cidx, ones_hbm, zeros1_hbm)
    inv_dr = _recip(counts_p)

    x0 = _proj(x_a, Wa, ba, x_b, Wb, bb)

    def bias_full(bias):
        return jnp.concatenate(
            [jnp.zeros((R * HID,), jnp.float32), bias]).reshape(1, _WCOLS)

    wf = _wprep3(jnp.stack([basis0, basis1, basis2]),
                 jnp.stack([comp0, comp1, comp2]),
                 jnp.stack([root0, root1, root2]))
    wf0, wf1, wf2 = wf[0], wf[1], wf[2]

    hz0 = _mm0(x0, wf0, bias_full(bias0))
    p0a, p0b = _agg_call(hz0.reshape(N_NODES * 9, HID), gidx, cidx, dst2,
                         inv_dr, zeros2_hbm)
    hz1 = _mmn(hz0, p0a, p0b, wf1, bias_full(bias1))
    p1a, p1b = _agg_call(hz1.reshape(N_NODES * 9, HID), gidx, cidx, dst2,
                         inv_dr, zeros2_hbm)
    hz2 = _mmn(hz1, p1a, p1b, wf2, bias_full(bias2))
    p2a, p2b = _agg_call(hz2.reshape(N_NODES * 9, HID), gidx, cidx, dst2,
                         inv_dr, zeros2_hbm)
    return _fin(hz2, p2a, p2b)
